# baseline (speedup 1.0000x reference)
import functools

import jax
import jax.numpy as jnp
from jax import lax
from jax.experimental import pallas as pl
from jax.experimental.pallas import tpu as pltpu
from jax.experimental.pallas import tpu_sc as plsc

NC = 2
NS = 16
NW = NC * NS
K = 16
NB = 5
LAG = 2
DSTB = 14

_GROUPS = [(0, range(0, 16))]


def _relmm_body(x_ref, w_ref, y_ref):
    for r in range(w_ref.shape[0]):
        y_ref[r] = jnp.dot(x_ref[...], w_ref[r],
                           preferred_element_type=jnp.float32)


def _final_body(x_ref, w0_ref, p_ref, o_ref):
    h = p_ref[0] + p_ref[1]
    acc = jnp.dot(x_ref[...], w0_ref[...], preferred_element_type=jnp.float32)
    o_ref[...] = jnp.maximum(acc + h, 0.0)


def _sc_body(epw, ch, n, d,
             y_hbm, pk_hbm, norm_hbm, zeros_hbm, out_hbm,
             acc, pk_v, norm_v, gi_v, di_v, *bufs):
    rows = bufs[0:NB]
    sg = bufs[NB:2 * NB]
    ss = bufs[2 * NB:3 * NB]
    c = lax.axis_index("c")
    s = lax.axis_index("s")
    w = c * NS + s

    pltpu.sync_copy(pk_hbm.at[w], pk_v)
    pltpu.sync_copy(norm_hbm.at[w], norm_v)

    rpt = (n // NS) & ~7
    tail = n - NS * rpt
    pltpu.sync_copy(zeros_hbm.at[pl.ds(s * rpt, rpt)],
                    acc.at[pl.ds(s * rpt, rpt)])
    if tail:
        @pl.when(s == NS - 1)
        def _zero_tail():
            pltpu.sync_copy(zeros_hbm.at[pl.ds(NS * rpt, tail)],
                            acc.at[pl.ds(NS * rpt, tail)])
    plsc.subcore_barrier()

    def unpack_gather_idx(j, b):
        for off, _ in _GROUPS:
            p = pk_v[pl.ds(j * K + off, 16)]
            gi_v[b, pl.ds(off, 16)] = lax.shift_right_logical(p, DSTB)

    def issue_gather(j, b):
        unpack_gather_idx(j, b)
        pltpu.async_copy(y_hbm.at[gi_v.at[b]], rows[b], sg[b])

    def wait_gather(b):
        pltpu.make_async_copy(y_hbm.at[gi_v.at[b]], rows[b], sg[b]).wait()

    def issue_scatter(b):
        pltpu.async_copy(rows[b], acc.at[di_v.at[b]], ss[b], add=True)

    def wait_scatter(b):
        pltpu.make_async_copy(rows[b], acc.at[di_v.at[b]], ss[b]).wait()

    def scale(j, b):
        for off, lanes in _GROUPS:
            p = pk_v[pl.ds(j * K + off, 16)]
            di_v[b, pl.ds(off, 16)] = p & ((1 << DSTB) - 1)
            nvec = norm_v[pl.ds(j * K + off, 16)]
            for i in lanes:
                nv = nvec[i]
                row = off + i
                for cg in range(d // 16):
                    sl = pl.ds(cg * 16, 16)
                    rows[b][row, sl] = rows[b][row, sl] * nv

    no = ch // NB
    for b in range(NB):
        issue_gather(b, b)

    def outer(o, carry):
        for i in range(NB):
            t = o * NB + i
            wait_gather(i)
            scale(t, i)
            issue_scatter(i)
            b2 = (i - LAG) % NB
            def rearm(b2=b2, t=t):
                wait_scatter(b2)
                issue_gather(t - LAG + NB, b2)
            if i < LAG:
                pl.when(o >= 1)(rearm)
            else:
                pl.when(o <= no - 2)(rearm)
        return carry

    lax.fori_loop(0, no, outer, 0, unroll=False)
    for b in range(NB):
        wait_scatter(b)
    plsc.subcore_barrier()

    pltpu.sync_copy(acc.at[pl.ds(s * rpt, rpt)],
                    out_hbm.at[c, pl.ds(s * rpt, rpt)])
    if tail:
        @pl.when(s == NS - 1)
        def _out_tail():
            pltpu.sync_copy(acc.at[pl.ds(NS * rpt, tail)],
                            out_hbm.at[c, pl.ds(NS * rpt, tail)])


def kernel(x, edge_index, edge_id, norm, weight, W_0):
    n, d = x.shape
    r = weight.shape[0]
    e = edge_index.shape[1]
    epw = e // NW
    ch = epw // K
    assert epw * NW == e and ch * K == epw and ch % NB == 0
    assert n <= (1 << DSTB) and r * n < (1 << (31 - DSTB))

    bn = 400
    gi = n // bn
    y = pl.pallas_call(
        _relmm_body,
        grid=(gi,),
        in_specs=[
            pl.BlockSpec((bn, d), lambda i: (i, 0)),
            pl.BlockSpec((r, d, d), lambda i: (0, 0, 0)),
        ],
        out_specs=pl.BlockSpec((r, bn, d), lambda i: (0, i, 0)),
        out_shape=jax.ShapeDtypeStruct((r, n, d), jnp.float32),
    )(x, weight)
    y_flat = y.reshape(r * n, d)

    src = edge_index[0]
    dst = edge_index[1]
    gidx = (edge_id * n + src).astype(jnp.int32)
    pk = ((gidx << DSTB) | dst.astype(jnp.int32)).reshape(NW, epw)
    norm_r = norm.astype(jnp.float32).reshape(NW, epw)
    zeros = jnp.zeros((n, d), jnp.float32)

    mesh = plsc.VectorSubcoreMesh(core_axis_name="c", subcore_axis_name="s")
    partials = pl.kernel(
        functools.partial(_sc_body, epw, ch, n, d),
        out_type=jax.ShapeDtypeStruct((NC, n, d), jnp.float32),
        mesh=mesh,
        scratch_types=(
            [
                pltpu.VMEM_SHARED((n, d), jnp.float32),
                pltpu.VMEM((epw,), jnp.int32),
                pltpu.VMEM((epw,), jnp.float32),
                pltpu.VMEM((NB, K), jnp.int32),
                pltpu.VMEM((NB, K), jnp.int32),
            ]
            + [pltpu.VMEM((K, d), jnp.float32) for _ in range(NB)]
            + [pltpu.SemaphoreType.DMA for _ in range(2 * NB)]
        ),
    )(y_flat, pk, norm_r, zeros)

    out = pl.pallas_call(
        _final_body,
        grid=(gi,),
        in_specs=[
            pl.BlockSpec((bn, d), lambda i: (i, 0)),
            pl.BlockSpec((d, d), lambda i: (0, 0)),
            pl.BlockSpec((NC, bn, d), lambda i: (0, i, 0)),
        ],
        out_specs=pl.BlockSpec((bn, d), lambda i: (i, 0)),
        out_shape=jax.ShapeDtypeStruct((n, d), jnp.float32),
    )(x, W_0, partials)
    return out


# speedup vs baseline: 1.1564x; 1.1564x over previous
import functools

import jax
import jax.numpy as jnp
from jax import lax
from jax.experimental import pallas as pl
from jax.experimental.pallas import tpu as pltpu
from jax.experimental.pallas import tpu_sc as plsc

NC = 2
NS = 16
NW = NC * NS
K = 16
NB = 5
LAG = 1
DSTB = 14

_GROUPS = [(0, range(0, 16))]


def _relmm_body(x_ref, w_ref, y_ref):
    for r in range(w_ref.shape[0]):
        y_ref[r] = jnp.dot(x_ref[...], w_ref[r],
                           preferred_element_type=jnp.float32)


def _final_body(x_ref, w0_ref, p_ref, o_ref):
    h = p_ref[0] + p_ref[1]
    acc = jnp.dot(x_ref[...], w0_ref[...], preferred_element_type=jnp.float32)
    o_ref[...] = jnp.maximum(acc + h, 0.0)


def _sc_body(epw, ch, n, d,
             y_hbm, pk_hbm, norm_hbm, zeros_hbm, out_hbm,
             acc, pk_v, norm_v, gi_v, di_v, *bufs):
    rows = bufs[0:NB]
    sg = bufs[NB:2 * NB]
    ss = bufs[2 * NB:3 * NB]
    c = lax.axis_index("c")
    s = lax.axis_index("s")
    w = c * NS + s

    pltpu.sync_copy(pk_hbm.at[w], pk_v)
    pltpu.sync_copy(norm_hbm.at[w], norm_v)

    rpt = (n // NS) & ~7
    tail = n - NS * rpt
    pltpu.sync_copy(zeros_hbm.at[pl.ds(s * rpt, rpt)],
                    acc.at[pl.ds(s * rpt, rpt)])
    if tail:
        @pl.when(s == NS - 1)
        def _zero_tail():
            pltpu.sync_copy(zeros_hbm.at[pl.ds(NS * rpt, tail)],
                            acc.at[pl.ds(NS * rpt, tail)])
    plsc.subcore_barrier()

    def unpack_gather_idx(j, b):
        for off, _ in _GROUPS:
            p = pk_v[pl.ds(j * K + off, 16)]
            gi_v[b, pl.ds(off, 16)] = lax.shift_right_logical(p, DSTB)

    def issue_gather(j, b):
        unpack_gather_idx(j, b)
        pltpu.async_copy(y_hbm.at[gi_v.at[b]], rows[b], sg[b])

    def wait_gather(b):
        pltpu.make_async_copy(y_hbm.at[gi_v.at[b]], rows[b], sg[b]).wait()

    def issue_scatter(b):
        pltpu.async_copy(rows[b], acc.at[di_v.at[b]], ss[b], add=True)

    def wait_scatter(b):
        pltpu.make_async_copy(rows[b], acc.at[di_v.at[b]], ss[b]).wait()

    def scale(j, b):
        for off, lanes in _GROUPS:
            p = pk_v[pl.ds(j * K + off, 16)]
            di_v[b, pl.ds(off, 16)] = p & ((1 << DSTB) - 1)
            nvec = norm_v[pl.ds(j * K + off, 16)]
            for i in lanes:
                nv = nvec[i]
                row = off + i
                for cg in range(d // 16):
                    sl = pl.ds(cg * 16, 16)
                    rows[b][row, sl] = rows[b][row, sl] * nv

    no = ch // NB
    for b in range(NB):
        issue_gather(b, b)

    def outer(o, carry):
        for i in range(NB):
            t = o * NB + i
            wait_gather(i)
            scale(t, i)
            issue_scatter(i)
            b2 = (i - LAG) % NB
            def rearm(b2=b2, t=t):
                wait_scatter(b2)
                issue_gather(t - LAG + NB, b2)
            if i < LAG:
                pl.when(o >= 1)(rearm)
            else:
                pl.when(o <= no - 2)(rearm)
        return carry

    lax.fori_loop(0, no, outer, 0, unroll=False)
    for b in range(NB):
        wait_scatter(b)
    plsc.subcore_barrier()

    pltpu.sync_copy(acc.at[pl.ds(s * rpt, rpt)],
                    out_hbm.at[c, pl.ds(s * rpt, rpt)])
    if tail:
        @pl.when(s == NS - 1)
        def _out_tail():
            pltpu.sync_copy(acc.at[pl.ds(NS * rpt, tail)],
                            out_hbm.at[c, pl.ds(NS * rpt, tail)])


def kernel(x, edge_index, edge_id, norm, weight, W_0):
    n, d = x.shape
    r = weight.shape[0]
    e = edge_index.shape[1]
    epw = e // NW
    ch = epw // K
    assert epw * NW == e and ch * K == epw and ch % NB == 0
    assert n <= (1 << DSTB) and r * n < (1 << (31 - DSTB))

    bn = 400
    gi = n // bn
    y = pl.pallas_call(
        _relmm_body,
        grid=(gi,),
        in_specs=[
            pl.BlockSpec((bn, d), lambda i: (i, 0)),
            pl.BlockSpec((r, d, d), lambda i: (0, 0, 0)),
        ],
        out_specs=pl.BlockSpec((r, bn, d), lambda i: (0, i, 0)),
        out_shape=jax.ShapeDtypeStruct((r, n, d), jnp.float32),
    )(x, weight)
    y_flat = y.reshape(r * n, d)

    src = edge_index[0]
    dst = edge_index[1]
    gidx = (edge_id * n + src).astype(jnp.int32)
    pk = ((gidx << DSTB) | dst.astype(jnp.int32)).reshape(NW, epw)
    norm_r = norm.astype(jnp.float32).reshape(NW, epw)
    zeros = jnp.zeros((n, d), jnp.float32)

    mesh = plsc.VectorSubcoreMesh(core_axis_name="c", subcore_axis_name="s")
    partials = pl.kernel(
        functools.partial(_sc_body, epw, ch, n, d),
        out_type=jax.ShapeDtypeStruct((NC, n, d), jnp.float32),
        mesh=mesh,
        scratch_types=(
            [
                pltpu.VMEM_SHARED((n, d), jnp.float32),
                pltpu.VMEM((epw,), jnp.int32),
                pltpu.VMEM((epw,), jnp.float32),
                pltpu.VMEM((NB, K), jnp.int32),
                pltpu.VMEM((NB, K), jnp.int32),
            ]
            + [pltpu.VMEM((K, d), jnp.float32) for _ in range(NB)]
            + [pltpu.SemaphoreType.DMA for _ in range(2 * NB)]
        ),
    )(y_flat, pk, norm_r, zeros)

    out = pl.pallas_call(
        _final_body,
        grid=(gi,),
        in_specs=[
            pl.BlockSpec((bn, d), lambda i: (i, 0)),
            pl.BlockSpec((d, d), lambda i: (0, 0)),
            pl.BlockSpec((NC, bn, d), lambda i: (0, i, 0)),
        ],
        out_specs=pl.BlockSpec((bn, d), lambda i: (i, 0)),
        out_shape=jax.ShapeDtypeStruct((n, d), jnp.float32),
    )(x, W_0, partials)
    return out
